# Initial kernel scaffold; baseline (speedup 1.0000x reference)
import jax
import jax.numpy as jnp
from jax.experimental import pallas as pl


def kernel(day, table):
    raise NotImplementedError("write your pallas kernel here")


# speedup vs baseline: 2.7585x; 2.7585x over previous
import functools

import jax
import jax.numpy as jnp
from jax import lax
from jax.experimental import pallas as pl
from jax.experimental.pallas import tpu as pltpu
from jax.experimental.pallas import tpu_sc as plsc

IDX_GROUP = 128
GROUPS = 4
CHUNK = IDX_GROUP * GROUPS


def _emb_kernel(n_per_w, n_chunks, embed, nc, day_hbm, table_hbm, out_hbm,
                idx_v, rows_v, sem):
    wid = lax.axis_index("s") * nc + lax.axis_index("c")
    w_base = wid * n_per_w

    def body(c, carry):
        base = w_base + c * CHUNK
        pltpu.sync_copy(day_hbm.at[pl.ds(base, CHUNK)], idx_v)
        copies = [
            pltpu.async_copy(
                table_hbm.at[idx_v.at[pl.ds(j * IDX_GROUP, IDX_GROUP)]],
                rows_v.at[pl.ds(j * IDX_GROUP, IDX_GROUP)],
                sem,
            )
            for j in range(GROUPS)
        ]
        for cp in copies:
            cp.wait()
        pltpu.sync_copy(rows_v, out_hbm.at[pl.ds(base, CHUNK)])
        return carry

    lax.fori_loop(0, n_chunks, body, 0)


def kernel(day, table):
    batch, hist = day.shape
    vocab, embed = table.shape
    n = batch * hist

    info = plsc.get_sparse_core_info()
    nc, ns = info.num_cores, info.num_subcores
    nw = nc * ns
    assert n % (nw * CHUNK) == 0
    n_per_w = n // nw
    n_chunks = n_per_w // CHUNK

    mesh = plsc.VectorSubcoreMesh(core_axis_name="c", subcore_axis_name="s")
    k = functools.partial(
        pl.kernel,
        mesh=mesh,
        out_type=jax.ShapeDtypeStruct((n, embed), jnp.float32),
        scratch_types=[
            pltpu.VMEM((CHUNK,), jnp.int32),
            pltpu.VMEM((CHUNK, embed), jnp.float32),
            pltpu.SemaphoreType.DMA,
        ],
        compiler_params=pltpu.CompilerParams(use_tc_tiling_on_sc=False),
    )(functools.partial(_emb_kernel, n_per_w, n_chunks, embed, nc))

    flat = k(day.reshape(n), table)
    return flat.reshape(batch, hist, embed)


# speedup vs baseline: 2.7745x; 1.0058x over previous
import functools

import jax
import jax.numpy as jnp
from jax import lax
from jax.experimental import pallas as pl
from jax.experimental.pallas import tpu as pltpu
from jax.experimental.pallas import tpu_sc as plsc

IDX_GROUP = 128
GROUPS = 5
CHUNK = IDX_GROUP * GROUPS


def _emb_kernel(n_per_w, n_chunks, embed, nc, day_hbm, table_hbm, out_hbm,
                idx_v, rows_v, gsem, osem):
    wid = lax.axis_index("s") * nc + lax.axis_index("c")
    w_base = wid * n_per_w

    def load_and_fire(c, b):
        base = w_base + c * CHUNK
        pltpu.sync_copy(day_hbm.at[pl.ds(base, CHUNK)], idx_v.at[b])
        for j in range(GROUPS):
            pltpu.async_copy(
                table_hbm.at[idx_v.at[b, pl.ds(j * IDX_GROUP, IDX_GROUP)]],
                rows_v.at[b, pl.ds(j * IDX_GROUP, IDX_GROUP)],
                gsem,
            )

    def drain_gathers(b):
        pltpu.make_async_copy(
            out_hbm.at[pl.ds(0, CHUNK)], rows_v.at[b], gsem).wait()

    def drain_write(b):
        pltpu.make_async_copy(
            rows_v.at[b], out_hbm.at[pl.ds(0, CHUNK)], osem).wait()

    load_and_fire(0, 0)

    def body(c, carry):
        b = lax.rem(c, 2)
        nb = 1 - b
        drain_gathers(b)

        @pl.when(c >= 1)
        def _():
            drain_write(nb)

        @pl.when(c + 1 < n_chunks)
        def _():
            load_and_fire(c + 1, nb)

        pltpu.async_copy(
            rows_v.at[b], out_hbm.at[pl.ds(w_base + c * CHUNK, CHUNK)], osem)
        return carry

    lax.fori_loop(0, n_chunks, body, 0)
    drain_write(lax.rem(n_chunks - 1, 2))


def kernel(day, table):
    batch, hist = day.shape
    vocab, embed = table.shape
    n = batch * hist

    info = plsc.get_sparse_core_info()
    nc, ns = info.num_cores, info.num_subcores
    nw = nc * ns
    assert n % (nw * CHUNK) == 0
    n_per_w = n // nw
    n_chunks = n_per_w // CHUNK

    mesh = plsc.VectorSubcoreMesh(core_axis_name="c", subcore_axis_name="s")
    k = functools.partial(
        pl.kernel,
        mesh=mesh,
        out_type=jax.ShapeDtypeStruct((n, embed), jnp.float32),
        scratch_types=[
            pltpu.VMEM((2, CHUNK), jnp.int32),
            pltpu.VMEM((2, CHUNK, embed), jnp.float32),
            pltpu.SemaphoreType.DMA,
            pltpu.SemaphoreType.DMA,
        ],
        compiler_params=pltpu.CompilerParams(use_tc_tiling_on_sc=False),
    )(functools.partial(_emb_kernel, n_per_w, n_chunks, embed, nc))

    flat = k(day.reshape(n), table)
    return flat.reshape(batch, hist, embed)
